# baseline (speedup 1.0000x reference)
import functools

import jax
import jax.numpy as jnp
from jax import lax
from jax.experimental import pallas as pl
from jax.experimental.pallas import tpu as pltpu
from jax.experimental.pallas import tpu_sc as plsc

DP = 128
NBUF = 4
K = 3
TBUF = 2
L = 16


def kernel(x, table):
    B, S = x.shape
    V, D = table.shape
    N = B * S

    info = plsc.get_sparse_core_info()
    NC, NS = info.num_cores, info.num_subcores
    NW = NC * NS
    CHUNK = 128
    per_w = N // NW
    n_chunks = per_w // CHUNK
    IB = B // CHUNK
    assert n_chunks % NBUF == 0 and K < NBUF

    table_p = jnp.pad(table, ((0, 0), (0, DP - D)))
    idx = x.T.reshape(NW, n_chunks, CHUNK)
    mesh = plsc.VectorSubcoreMesh(core_axis_name="c", subcore_axis_name="s")

    @functools.partial(
        pl.kernel,
        mesh=mesh,
        out_type=jax.ShapeDtypeStruct((D, S, B), jnp.float32),
        scratch_types=[
            pltpu.VMEM((n_chunks, CHUNK), jnp.int32),
            pltpu.VMEM((NBUF, CHUNK, DP), jnp.float32),
            pltpu.VMEM((TBUF, D, 1, CHUNK), jnp.float32),
        ]
        + [pltpu.SemaphoreType.DMA] * (NBUF + TBUF),
        compiler_params=pltpu.CompilerParams(use_tc_tiling_on_sc=True,
                                             needs_layout_passes=False),
    )
    def emb(idx_hbm, table_hbm, out_hbm, idx_v, rows_v, tp_v, *sems):
        gsem, wsem = sems[:NBUF], sems[NBUF:]
        wid = lax.axis_index("s") * NC + lax.axis_index("c")
        q0 = wid * n_chunks
        pltpu.sync_copy(idx_hbm.at[wid], idx_v)
        lanes = lax.iota(jnp.int32, L)

        def gather(c, b):
            return pltpu.make_async_copy(
                table_hbm.at[idx_v.at[c]], rows_v.at[b], gsem[b])

        def write(c, t):
            q = q0 + c
            return pltpu.make_async_copy(
                tp_v.at[t],
                out_hbm.at[:, pl.ds(q // IB, 1),
                           pl.ds((q % IB) * CHUNK, CHUNK)],
                wsem[t])

        for j in range(K):
            gather(j, j).start()

        def transpose(b, t):
            def tr(kk, carry):
                for g in range(CHUNK // L):
                    col = jnp.full((L,), kk, dtype=jnp.int32)
                    v = plsc.load_gather(rows_v.at[b],
                                         [lanes + g * L, col])
                    tp_v[t, kk, 0, pl.ds(g * L, L)] = v
                return carry
            lax.fori_loop(0, D, tr, 0)

        def group(gi, carry):
            for b in range(NBUF):
                c = gi * NBUF + b
                t = b % TBUF

                @pl.when(c < n_chunks - K)
                def _issue():
                    gather(c + K, (b + K) % NBUF).start()

                gather(c, b).wait()

                @pl.when(c >= TBUF)
                def _release():
                    write(c - TBUF, t).wait()

                transpose(b, t)
                write(c, t).start()
            return carry

        lax.fori_loop(0, n_chunks // NBUF, group, 0)
        for j in range(TBUF):
            c = n_chunks - TBUF + j
            write(c, c % TBUF).wait()

    out_t = emb(idx, table_p)
    return jnp.transpose(out_t, (2, 1, 0))


# speedup vs baseline: 1.5040x; 1.5040x over previous
import functools

import jax
import jax.numpy as jnp
from jax import lax
from jax.experimental import pallas as pl
from jax.experimental.pallas import tpu as pltpu
from jax.experimental.pallas import tpu_sc as plsc

DP = 128
NBUF = 4
K = 3
TBUF = 2
L = 16


def kernel(x, table):
    B, S = x.shape
    V, D = table.shape
    N = B * S

    info = plsc.get_sparse_core_info()
    NC, NS = info.num_cores, info.num_subcores
    NW = NC * NS
    CHUNK = 128
    per_w = N // NW
    n_chunks = per_w // CHUNK
    IB = B // CHUNK
    assert n_chunks % NBUF == 0 and K < NBUF

    table_p = jnp.pad(table, ((0, 0), (0, DP - D)))
    idx = x.T.reshape(NW, n_chunks, CHUNK)
    mesh = plsc.VectorSubcoreMesh(core_axis_name="c", subcore_axis_name="s")

    @functools.partial(
        pl.kernel,
        mesh=mesh,
        out_type=jax.ShapeDtypeStruct((D, S, B), jnp.float32),
        scratch_types=[
            pltpu.VMEM((n_chunks, CHUNK), jnp.int32),
            pltpu.VMEM((NBUF, CHUNK, DP), jnp.float32),
            pltpu.VMEM((TBUF, D, 1, CHUNK), jnp.float32),
        ]
        + [pltpu.SemaphoreType.DMA] * (NBUF + TBUF),
        compiler_params=pltpu.CompilerParams(use_tc_tiling_on_sc=True,
                                             needs_layout_passes=False),
    )
    def emb(idx_hbm, table_hbm, out_hbm, idx_v, rows_v, tp_v, *sems):
        gsem, wsem = sems[:NBUF], sems[NBUF:]
        wid = lax.axis_index("s") * NC + lax.axis_index("c")
        q0 = wid * n_chunks
        pltpu.sync_copy(idx_hbm.at[wid], idx_v)
        lanes = lax.iota(jnp.int32, L)

        def gather(c, b):
            return pltpu.make_async_copy(
                table_hbm.at[idx_v.at[c]], rows_v.at[b], gsem[b])

        def write(c, t):
            q = q0 + c
            return pltpu.make_async_copy(
                tp_v.at[t],
                out_hbm.at[:, pl.ds(q // IB, 1),
                           pl.ds((q % IB) * CHUNK, CHUNK)],
                wsem[t])

        for j in range(K):
            gather(j, j).start()

        def transpose(b, t):
            @plsc.parallel_loop(0, D, unroll=4)
            def tr(kk):
                for g in range(CHUNK // L):
                    col = jnp.full((L,), kk, dtype=jnp.int32)
                    v = plsc.load_gather(rows_v.at[b],
                                         [lanes + g * L, col])
                    tp_v[t, kk, 0, pl.ds(g * L, L)] = v

        def group(gi, carry):
            for b in range(NBUF):
                c = gi * NBUF + b
                t = b % TBUF

                @pl.when(c < n_chunks - K)
                def _issue():
                    gather(c + K, (b + K) % NBUF).start()

                gather(c, b).wait()

                @pl.when(c >= TBUF)
                def _release():
                    write(c - TBUF, t).wait()

                transpose(b, t)
                write(c, t).start()
            return carry

        lax.fori_loop(0, n_chunks // NBUF, group, 0)
        for j in range(TBUF):
            c = n_chunks - TBUF + j
            write(c, c % TBUF).wait()

    out_t = emb(idx, table_p)
    return jnp.transpose(out_t, (2, 1, 0))


# speedup vs baseline: 1.9961x; 1.3272x over previous
import functools

import jax
import jax.numpy as jnp
from jax import lax
from jax.experimental import pallas as pl
from jax.experimental.pallas import tpu as pltpu
from jax.experimental.pallas import tpu_sc as plsc

DP = 128
NBUF = 5
K = 2


def kernel(x, table):
    B, S = x.shape
    V, D = table.shape
    N = B * S

    info = plsc.get_sparse_core_info()
    NC, NS = info.num_cores, info.num_subcores
    NW = NC * NS
    CHUNK = 128
    per_w = N // NW
    n_chunks = per_w // CHUNK
    assert n_chunks % NBUF == 0 and K < NBUF

    table_p = jnp.pad(table, ((0, 0), (0, DP - D)))
    idx = x.reshape(NW, n_chunks, CHUNK)
    mesh = plsc.VectorSubcoreMesh(core_axis_name="c", subcore_axis_name="s")

    @functools.partial(
        pl.kernel,
        mesh=mesh,
        out_type=jax.ShapeDtypeStruct((NW, per_w, DP), jnp.float32),
        scratch_types=[
            pltpu.VMEM((n_chunks, CHUNK), jnp.int32),
            pltpu.VMEM((NBUF, CHUNK, DP), jnp.float32),
        ]
        + [pltpu.SemaphoreType.DMA] * (2 * NBUF),
        compiler_params=pltpu.CompilerParams(use_tc_tiling_on_sc=True),
    )
    def emb(idx_hbm, table_hbm, out_hbm, idx_v, rows_v, *sems):
        gsem, wsem = sems[:NBUF], sems[NBUF:]
        wid = lax.axis_index("s") * NC + lax.axis_index("c")
        pltpu.sync_copy(idx_hbm.at[wid], idx_v)

        def gather(c, b, sem):
            return pltpu.make_async_copy(
                table_hbm.at[idx_v.at[c]], rows_v.at[b], sem)

        def write(c, b, sem):
            return pltpu.make_async_copy(
                rows_v.at[b], out_hbm.at[wid, pl.ds(c * CHUNK, CHUNK)], sem)

        for j in range(K):
            gather(j, j, gsem[j]).start()

        def group(g, carry):
            for b in range(NBUF):
                c = g * NBUF + b
                bk = (b + K) % NBUF

                @pl.when(c < n_chunks - K)
                def _issue():
                    @pl.when(c >= NBUF - K)
                    def _release():
                        write(c + K - NBUF, bk, wsem[bk]).wait()

                    gather(c + K, bk, gsem[bk]).start()

                gather(c, b, gsem[b]).wait()
                write(c, b, wsem[b]).start()
            return carry

        lax.fori_loop(0, n_chunks // NBUF, group, 0)
        for j in range(NBUF):
            c = n_chunks - NBUF + j
            write(c, c % NBUF, wsem[c % NBUF]).wait()

    out = emb(idx, table_p)
    return out.reshape(N, DP)[:, :D].reshape(B, S, D)
